# Initial kernel scaffold; baseline (speedup 1.0000x reference)
import jax
import jax.numpy as jnp
from jax.experimental import pallas as pl


def kernel(x, gate_w, w1, w3, w2):
    raise NotImplementedError("write your pallas kernel here")


# speedup vs baseline: 1.0854x; 1.0854x over previous
import jax
import jax.numpy as jnp
from jax.experimental import pallas as pl
from jax.experimental.pallas import tpu as pltpu

H = 2048
FF = 4096
E = 8
B = 32
BF = 512
NF = FF // BF


def _moe_body(x_ref, gate_w_ref, w1_ref, w3_ref, w2_ref, out_ref, wts_ref):
    e = pl.program_id(0)
    f = pl.program_id(1)
    first = (e == 0) & (f == 0)

    @pl.when(first)
    def _gate():
        xb = x_ref[...]
        logits = jnp.dot(xb, gate_w_ref[...],
                         preferred_element_type=jnp.float32)
        neg_inf = jnp.finfo(jnp.float32).min
        ex0 = jnp.max(logits, axis=1, keepdims=True)
        cond0 = (logits == ex0).astype(jnp.float32)
        masked = jnp.where(logits == ex0, neg_inf, logits)
        ex1 = jnp.max(masked, axis=1, keepdims=True)
        cond1 = (masked == ex1).astype(jnp.float32)
        w_pre = 1.0 / (1.0 + jnp.exp(ex1 - ex0))
        wts_ref[...] = cond0 * w_pre - cond1 * (w_pre - 1.0)

    xb = x_ref[...]
    h1 = jnp.dot(xb, w1_ref[0], preferred_element_type=jnp.float32)
    h3 = jnp.dot(xb, w3_ref[0], preferred_element_type=jnp.float32)
    g = (h1 * jax.nn.sigmoid(h1)) * h3
    wts = wts_ref[...]
    lane = jax.lax.broadcasted_iota(jnp.int32, (B, E), 1)
    wcol = jnp.sum(jnp.where(lane == e, wts, 0.0), axis=1, keepdims=True)
    g = g * wcol
    partial = jnp.dot(g, w2_ref[0], preferred_element_type=jnp.float32)

    @pl.when(first)
    def _init():
        out_ref[...] = partial

    @pl.when(~first)
    def _acc():
        out_ref[...] += partial


def kernel(x, gate_w, w1, w3, w2):
    xb = x.reshape(B, H)
    out = pl.pallas_call(
        _moe_body,
        grid=(E, NF),
        in_specs=[
            pl.BlockSpec((B, H), lambda e, f: (0, 0)),
            pl.BlockSpec((H, E), lambda e, f: (0, 0)),
            pl.BlockSpec((1, H, BF), lambda e, f: (e, 0, f)),
            pl.BlockSpec((1, H, BF), lambda e, f: (e, 0, f)),
            pl.BlockSpec((1, BF, H), lambda e, f: (e, f, 0)),
        ],
        out_specs=pl.BlockSpec((B, H), lambda e, f: (0, 0)),
        out_shape=jax.ShapeDtypeStruct((B, H), jnp.float32),
        scratch_shapes=[pltpu.VMEM((B, E), jnp.float32)],
        compiler_params=pltpu.CompilerParams(
            dimension_semantics=("arbitrary", "arbitrary"),
        ),
    )(xb, gate_w, w1, w3, w2)
    return out.reshape(1, 1, B, H)
